# baseline (speedup 1.0000x reference)
import functools

import jax
import jax.numpy as jnp
from jax import lax
from jax.experimental import pallas as pl
from jax.experimental.pallas import tpu as pltpu
from jax.experimental.pallas import tpu_sc as plsc

B = 16384
F = 26
CARD = 100000
D = 32
EPS = 1e-5
ROW = CARD + 1

FH = F // 2
PH = FH * D

NC = 2
NS = 16
NW = NC * NS
P_PER_W = PH // NW
HALF = B // 2


def _sc_gather(planes, idxT, f_off):
    mesh = plsc.VectorSubcoreMesh(core_axis_name="c", subcore_axis_name="s")

    @functools.partial(
        pl.kernel,
        mesh=mesh,
        compiler_params=pltpu.CompilerParams(use_tc_tiling_on_sc=True,
                                             needs_layout_passes=False),
        out_type=jax.ShapeDtypeStruct((PH, B), jnp.float32),
        scratch_types=[
            pltpu.VMEM((ROW,), jnp.float32),
            pltpu.VMEM((HALF,), jnp.int32),
            pltpu.VMEM((HALF,), jnp.float32),
        ],
        name=f"sc_gather_f{f_off}",
    )
    def k(pl_hbm, idx_hbm, out_hbm, plane_v, idx_v, out_v):
        wid = lax.axis_index("s") * NC + lax.axis_index("c")
        q0 = wid * P_PER_W

        def plane_body(t, _):
            q = q0 + t
            f = f_off + q // D
            pltpu.sync_copy(pl_hbm.at[f_off * D + q], plane_v)

            def half_body(h, _):
                pltpu.sync_copy(idx_hbm.at[f, pl.ds(h * HALF, HALF)], idx_v)

                def group_body(g, _):
                    i16 = idx_v[pl.ds(g * 16, 16)]
                    out_v[pl.ds(g * 16, 16)] = plsc.load_gather(plane_v, [i16])
                    return 0

                lax.fori_loop(0, HALF // 16, group_body, 0)
                pltpu.sync_copy(out_v, out_hbm.at[q, pl.ds(h * HALF, HALF)])
                return 0

            lax.fori_loop(0, 2, half_body, 0)
            return 0

        lax.fori_loop(0, P_PER_W, plane_body, 0)

    return k(planes, idxT)


BT = 1024


def _tc_body(et_ref, w0, w1, b_ref, g_ref, bt_ref, s_ref, e_ref, out_ref):
    hi = jax.lax.Precision.DEFAULT
    dn = (((0,), (0,)), ((), ()))
    et = et_ref[...]
    c0 = lax.dot_general(et[0:256, :], w0[...], dn, precision=hi)
    c1 = lax.dot_general(et[256:PH, :], w1[...], dn, precision=hi)
    c = jnp.concatenate([c0, c1], axis=1) + b_ref[...]
    sq = c * c
    msq = jnp.dot(sq, s_ref[...], precision=hi)
    r = lax.rsqrt(msq + EPS)
    scale = jnp.dot(r, e_ref[...], precision=hi)
    out_ref[...] = c * scale * g_ref[...] + bt_ref[...]


def _tc_norm(embT, w0, w1, b416, g416, bt416, S, E):
    grid = (B // BT,)
    full = lambda shape: pl.BlockSpec(shape, lambda i: (0, 0))
    return pl.pallas_call(
        _tc_body,
        grid=grid,
        in_specs=[
            pl.BlockSpec((PH, BT), lambda i: (0, i)),
            full((256, 256)), full((PH - 256, PH - 256)),
            full((1, PH)), full((1, PH)), full((1, PH)),
            full((PH, 128)), full((128, PH)),
        ],
        out_specs=pl.BlockSpec((BT, PH), lambda i: (i, 0)),
        out_shape=jax.ShapeDtypeStruct((B, PH), jnp.float32),
    )(embT, w0, w1, b416, g416, bt416, S, E)


def kernel(x, tables, proj_W, proj_b, gamma, beta):
    idxT = jnp.clip(x, 0, CARD).astype(jnp.int32).T
    planes = tables.transpose(0, 2, 1).reshape(F * D, ROW)

    C = jnp.eye(D, dtype=jnp.float32) - jnp.full((D, D), 1.0 / D,
                                                 dtype=jnp.float32)
    Wc = jnp.matmul(proj_W, C)
    bc = jnp.matmul(proj_b, C)

    d_ids = jnp.arange(PH, dtype=jnp.int32) // D
    S = (d_ids[:, None] == jnp.arange(128, dtype=jnp.int32)[None, :]
         ).astype(jnp.float32) / D
    E = (jnp.arange(128, dtype=jnp.int32)[:, None] == d_ids[None, :]
         ).astype(jnp.float32)

    blkdiag = jax.scipy.linalg.block_diag
    halves = []
    for hf in range(2):
        f0 = hf * FH
        w0 = blkdiag(*[Wc[f] for f in range(f0, f0 + 8)])
        w1 = blkdiag(*[Wc[f] for f in range(f0 + 8, f0 + FH)])
        b416 = bc[f0:f0 + FH].reshape(1, PH)
        g416 = jnp.tile(gamma, FH)[None, :]
        bt416 = jnp.tile(beta, FH)[None, :]
        embT = _sc_gather(planes, idxT, f0)
        o = _tc_norm(embT, w0, w1, b416, g416, bt416, S, E)
        halves.append(o.reshape(B, FH, D))
    return jnp.concatenate(halves, axis=1)
